# baseline (speedup 1.0000x reference)
import functools

import jax
import jax.numpy as jnp
from jax import lax
from jax.experimental import pallas as pl
from jax.experimental.pallas import tpu as pltpu
from jax.experimental.pallas import tpu_sc as plsc

NC = 2
NS = 16
NW = NC * NS
LANES = 16
CH = 128
NBUF = 4


@functools.partial(jax.jit, static_argnames=("n_chunks", "d"))
def _embed_sum(x3, W0, W1, n_chunks, d):
    total = NW * n_chunks * CH
    n_groups = n_chunks // NBUF
    mesh = plsc.VectorSubcoreMesh(
        core_axis_name="c", subcore_axis_name="s",
        num_cores=NC, num_subcores=NS)

    @functools.partial(
        pl.kernel,
        mesh=mesh,
        compiler_params=pltpu.CompilerParams(use_tc_tiling_on_sc=False),
        out_type=jax.ShapeDtypeStruct((total, d), jnp.float32),
        scratch_types=[
            pltpu.VMEM((n_chunks, CH), jnp.int32),
            pltpu.VMEM((NBUF, CH, d), jnp.float32),
            pltpu.VMEM((NBUF, CH, d), jnp.float32),
            pltpu.VMEM((NBUF, CH, d), jnp.float32),
            pltpu.SemaphoreType.DMA((NBUF,)),
            pltpu.SemaphoreType.DMA((NBUF,)),
        ],
    )
    def body(x_hbm, w0_hbm, w1_hbm, out_hbm, idx_v, r0, r1, o, semg, semo):
        wid = lax.axis_index("s") * NC + lax.axis_index("c")
        pltpu.sync_copy(x_hbm.at[wid], idx_v)
        out_base = wid * n_chunks * CH

        for b in range(NBUF):
            pltpu.async_copy(w0_hbm.at[idx_v.at[b]], r0.at[b], semg.at[b])
            pltpu.async_copy(w1_hbm.at[idx_v.at[b]], r1.at[b], semg.at[b])

        def group_body(g, carry):
            for b in range(NBUF):
                i = g * NBUF + b
                pltpu.make_async_copy(
                    w0_hbm.at[idx_v.at[i]], r0.at[b], semg.at[b]).wait()
                pltpu.make_async_copy(
                    w0_hbm.at[idx_v.at[i]], r1.at[b], semg.at[b]).wait()

                @pl.when(g > 0)
                def _():
                    pltpu.make_async_copy(
                        o.at[b], out_hbm.at[pl.ds(0, CH)], semo.at[b]).wait()

                def addloop(j, c2):
                    o[b, j, pl.ds(0, LANES)] = (
                        r0[b, j, pl.ds(0, LANES)] + r1[b, j, pl.ds(0, LANES)])
                    o[b, j, pl.ds(LANES, LANES)] = (
                        r0[b, j, pl.ds(LANES, LANES)]
                        + r1[b, j, pl.ds(LANES, LANES)])
                    return c2

                lax.fori_loop(0, CH, addloop, 0)

                @pl.when(i + NBUF < n_chunks)
                def _():
                    pltpu.async_copy(
                        w0_hbm.at[idx_v.at[i + NBUF]], r0.at[b], semg.at[b])
                    pltpu.async_copy(
                        w1_hbm.at[idx_v.at[i + NBUF]], r1.at[b], semg.at[b])

                pltpu.async_copy(
                    o.at[b], out_hbm.at[pl.ds(out_base + i * CH, CH)],
                    semo.at[b])
            return carry

        lax.fori_loop(0, n_groups, group_body, 0)

        for b in range(NBUF):
            pltpu.make_async_copy(
                o.at[b], out_hbm.at[pl.ds(0, CH)], semo.at[b]).wait()

    return body(x3, W0, W1)


def kernel(x, W0, W1):
    b, f = x.shape
    d = W0.shape[1]
    total = b * f
    assert total % (NW * CH * NBUF) == 0
    n_chunks = total // (NW * CH)
    x3 = x.reshape(NW, n_chunks, CH)
    out = _embed_sum(x3, W0, W1, n_chunks, d)
    return out.reshape(b, f, d)
